# baseline (speedup 1.0000x reference)
import functools

import jax
import jax.numpy as jnp
from jax.experimental import pallas as pl
from jax.experimental.pallas import tpu as pltpu

_D = 128
_W = 2 * _D + 1


def _predict_kernel(h_ref, wpw_ref, wpb_ref, vpwp_ref, vpb_ref,
                    s0_ref, sh_ref, p_ref, *, seq_len):
    h = h_ref[...]
    wph = jax.lax.dot_general(h, wpw_ref[...], (((1,), (1,)), ((), ())),
                              precision=jax.lax.Precision.DEFAULT)
    wph = jnp.tanh(wph + wpb_ref[...])
    logit = jax.lax.dot_general(wph, vpwp_ref[...], (((1,), (0,)), ((), ())),
                                precision=jax.lax.Precision.DEFAULT)
    p = seq_len * jax.nn.sigmoid(logit + vpb_ref[0, 0])
    center = jnp.round(p).astype(jnp.int32)
    u = center - _D
    s0 = jnp.clip(u, 0, seq_len - _F)
    s0_ref[...] = s0
    sh_ref[...] = u - s0
    p_ref[...] = p


_F = 264
_NBUF = 8
_CHUNKS = ((0, 64), (64, 64), (128, 64), (192, 72))


def _attn_kernel(s0_ref, sh_ref, p_ref, h_ref, enc_ref,
                 w_ref, ctx_ref, buf_ref, sem_ref, pad_ref, *, num_b):
    b = pl.program_id(0)

    @pl.when(b == 0)
    def _():
        pad_ref[...] = jnp.zeros_like(pad_ref)

    def dmas(i, slot):
        s0i = s0_ref[i]
        out = []
        for c, (lo, n) in enumerate(_CHUNKS):
            out.append(pltpu.make_async_copy(
                enc_ref.at[pl.ds(s0i + lo, n), i],
                buf_ref.at[slot, pl.ds(lo, n)],
                sem_ref.at[slot, c]))
        return out

    def start(i):
        for d in dmas(i, jax.lax.rem(i, _NBUF)):
            d.start()

    @pl.when(b == 0)
    def _():
        for i in range(_NBUF - 1):
            if i < num_b:
                start(i)

    @pl.when(b + _NBUF - 1 < num_b)
    def _():
        start(b + _NBUF - 1)

    slot = jax.lax.rem(b, _NBUF)
    for d in dmas(b, slot):
        d.wait()
    enc = buf_ref[slot]

    sh = sh_ref[b]
    s0 = s0_ref[b]
    pb = p_ref[b]
    scores = jax.lax.dot_general(h_ref[0], enc, (((1,), (1,)), ((), ())),
                                 precision=jax.lax.Precision.DEFAULT)
    r = jax.lax.broadcasted_iota(jnp.int32, (1, _F), 1)
    mask = (r >= sh) & (r < _W + sh)
    sm = jnp.where(mask, scores, -1e9)
    m = jnp.max(sm)
    e = jnp.where(mask, jnp.exp(sm - m), 0.0)
    wv = e / jnp.sum(e)
    a = (s0 + r).astype(jnp.float32)
    gauss = jnp.exp(-((a - pb) ** 2) * (1.0 / 8192.0))
    wsc = wv * gauss
    ctx_ref[0] = jax.lax.dot_general(wsc, enc, (((1,), (0,)), ((), ())),
                                     precision=jax.lax.Precision.DEFAULT)
    pad_ref[:, 128:128 + _F] = wsc
    rolled = pltpu.roll(pad_ref[...], -(128 + sh), 1)
    w_ref[0] = rolled[:, :_W]


def kernel(t, hidden, encoder_outputs, Wp_w, Wp_b, vp_w, vp_b):
    del t
    seq_len, num_b, d = encoder_outputs.shape

    vp_w_pad = jnp.zeros((d, 128), jnp.float32).at[:, 0].set(vp_w[0])

    s0, sh, p = pl.pallas_call(
        functools.partial(_predict_kernel, seq_len=seq_len),
        in_specs=[
            pl.BlockSpec((num_b, d), lambda: (0, 0)),
            pl.BlockSpec((d, d), lambda: (0, 0)),
            pl.BlockSpec((1, d), lambda: (0, 0)),
            pl.BlockSpec((d, 128), lambda: (0, 0)),
            pl.BlockSpec(memory_space=pltpu.SMEM),
        ],
        out_shape=[
            jax.ShapeDtypeStruct((num_b, 128), jnp.int32),
            jax.ShapeDtypeStruct((num_b, 128), jnp.int32),
            jax.ShapeDtypeStruct((num_b, 128), jnp.float32),
        ],
    )(hidden, Wp_w, Wp_b.reshape(1, d), vp_w_pad, vp_b.reshape(1, 1))

    grid_spec = pltpu.PrefetchScalarGridSpec(
        num_scalar_prefetch=3,
        grid=(num_b,),
        in_specs=[
            pl.BlockSpec((1, 1, d), lambda b, *_: (b, 0, 0)),
            pl.BlockSpec(memory_space=pl.ANY),
        ],
        out_specs=[
            pl.BlockSpec((1, 1, _W), lambda b, *_: (b, 0, 0)),
            pl.BlockSpec((1, 1, d), lambda b, *_: (b, 0, 0)),
        ],
        scratch_shapes=[
            pltpu.VMEM((_NBUF, _F, d), jnp.float32),
            pltpu.SemaphoreType.DMA((_NBUF, len(_CHUNKS))),
            pltpu.VMEM((1, 512), jnp.float32),
        ],
    )
    w_scaled, context = pl.pallas_call(
        functools.partial(_attn_kernel, num_b=num_b),
        grid_spec=grid_spec,
        out_shape=[
            jax.ShapeDtypeStruct((num_b, 1, _W), jnp.float32),
            jax.ShapeDtypeStruct((num_b, 1, d), jnp.float32),
        ],
    )(s0[:, 0], sh[:, 0], p[:, 0],
      hidden.reshape(num_b, 1, d), encoder_outputs)
    return (w_scaled.reshape(num_b, _W), context.reshape(num_b, d))


# speedup vs baseline: 1.2095x; 1.2095x over previous
import functools

import jax
import jax.numpy as jnp
from jax.experimental import pallas as pl
from jax.experimental.pallas import tpu as pltpu

_D = 128
_W = 2 * _D + 1
_F = 264
_NBUF = 8


def _fused_kernel(hfull_ref, wpw_ref, wpb_ref, vpw_ref, vpb_ref, h_ref,
                  enc_ref, w_ref, ctx_ref, buf_ref, sem_ref, pad_ref,
                  ivec_ref, fvec_ref, ismem_ref, fsmem_ref, ssem_ref,
                  *, seq_len, num_b):
    b = pl.program_id(0)

    @pl.when(b == 0)
    def _():
        pad_ref[...] = jnp.zeros_like(pad_ref)
        h = hfull_ref[...]
        wph = jax.lax.dot_general(h, wpw_ref[...], (((1,), (1,)), ((), ())),
                                  precision=jax.lax.Precision.DEFAULT)
        wph = jnp.tanh(wph + wpb_ref[...])
        logit = jax.lax.dot_general(vpw_ref[...], wph, (((1,), (1,)), ((), ())),
                                    precision=jax.lax.Precision.DEFAULT)
        p = seq_len * jax.nn.sigmoid(logit + vpb_ref[0, 0])
        center = jnp.round(p).astype(jnp.int32)
        u = center - _D
        s0 = jnp.clip(u, 0, seq_len - _F)
        ivec_ref[0:1] = s0
        ivec_ref[1:2] = u - s0
        fvec_ref[...] = p
        pltpu.make_async_copy(ivec_ref, ismem_ref, ssem_ref).start()
        pltpu.make_async_copy(ivec_ref, ismem_ref, ssem_ref).wait()
        pltpu.make_async_copy(fvec_ref, fsmem_ref, ssem_ref).start()
        pltpu.make_async_copy(fvec_ref, fsmem_ref, ssem_ref).wait()

    def dma(i, slot):
        return pltpu.make_async_copy(
            enc_ref.at[pl.ds(ismem_ref[0, i], _F), i],
            buf_ref.at[slot],
            sem_ref.at[slot])

    def start(i):
        dma(i, jax.lax.rem(i, _NBUF)).start()

    @pl.when(b == 0)
    def _():
        for i in range(min(_NBUF - 1, num_b)):
            start(i)

    @pl.when(b + _NBUF - 1 < num_b)
    def _():
        start(b + _NBUF - 1)

    slot = jax.lax.rem(b, _NBUF)
    dma(b, slot).wait()
    enc = buf_ref[slot]

    s0 = ismem_ref[0, b]
    sh = ismem_ref[1, b]
    pb = fsmem_ref[0, b]
    scores = jax.lax.dot_general(h_ref[0], enc, (((1,), (1,)), ((), ())),
                                 precision=jax.lax.Precision.DEFAULT)
    r = jax.lax.broadcasted_iota(jnp.int32, (1, _F), 1)
    mask = (r >= sh) & (r < _W + sh)
    sm = jnp.where(mask, scores, -1e9)
    m = jnp.max(sm)
    e = jnp.where(mask, jnp.exp(sm - m), 0.0)
    wv = e / jnp.sum(e)
    a = (s0 + r).astype(jnp.float32)
    gauss = jnp.exp(-((a - pb) ** 2) * (1.0 / 8192.0))
    wsc = wv * gauss
    ctx_ref[0] = jax.lax.dot_general(wsc, enc, (((1,), (0,)), ((), ())),
                                     precision=jax.lax.Precision.DEFAULT)
    pad_ref[:, 128:128 + _F] = wsc
    rolled = pltpu.roll(pad_ref[...], -(128 + sh), 1)
    w_ref[0] = rolled[:, :_W]


def kernel(t, hidden, encoder_outputs, Wp_w, Wp_b, vp_w, vp_b):
    del t
    seq_len, num_b, d = encoder_outputs.shape

    w_scaled, context = pl.pallas_call(
        functools.partial(_fused_kernel, seq_len=seq_len, num_b=num_b),
        grid=(num_b,),
        in_specs=[
            pl.BlockSpec((num_b, d), lambda b: (0, 0)),
            pl.BlockSpec((d, d), lambda b: (0, 0)),
            pl.BlockSpec((1, d), lambda b: (0, 0)),
            pl.BlockSpec((1, d), lambda b: (0, 0)),
            pl.BlockSpec(memory_space=pltpu.SMEM),
            pl.BlockSpec((1, 1, d), lambda b: (b, 0, 0)),
            pl.BlockSpec(memory_space=pl.ANY),
        ],
        out_specs=[
            pl.BlockSpec((1, 1, _W), lambda b: (b, 0, 0)),
            pl.BlockSpec((1, 1, d), lambda b: (b, 0, 0)),
        ],
        scratch_shapes=[
            pltpu.VMEM((_NBUF, _F, d), jnp.float32),
            pltpu.SemaphoreType.DMA((_NBUF,)),
            pltpu.VMEM((1, 512), jnp.float32),
            pltpu.VMEM((2, num_b), jnp.int32),
            pltpu.VMEM((1, num_b), jnp.float32),
            pltpu.SMEM((2, num_b), jnp.int32),
            pltpu.SMEM((1, num_b), jnp.float32),
            pltpu.SemaphoreType.DMA,
        ],
        out_shape=[
            jax.ShapeDtypeStruct((num_b, 1, _W), jnp.float32),
            jax.ShapeDtypeStruct((num_b, 1, d), jnp.float32),
        ],
    )(hidden, Wp_w, Wp_b.reshape(1, d), vp_w, vp_b.reshape(1, 1),
      hidden.reshape(num_b, 1, d), encoder_outputs)
    return (w_scaled.reshape(num_b, _W), context.reshape(num_b, d))
